# baseline (speedup 1.0000x reference)
import functools

import numpy as np

import jax
import jax.numpy as jnp
from jax import lax
from jax.experimental import pallas as pl
from jax.experimental.pallas import tpu as pltpu
from jax.experimental.pallas import tpu_sc as plsc

F32 = np.float32
I32 = np.int32

_I = 3
_S = 8
_A = 128
_T = 128
_NSUB = 16
_SPW = 32
_NROW = 24

_SCALE = F32(2.0 ** 60)
_LN2_120 = F32(120 * 0.6931471805599453)


def _iota16():
    return lax.iota(I32, 16)


def _perm(v, idx):
    return v.at[idx].get(mode="promise_in_bounds")


def _allsum(v, iota):
    for d in (1, 2, 4, 8):
        v = v + _perm(v, iota ^ d)
    return v


def _allmax(v, iota):
    for d in (1, 2, 4, 8):
        v = jnp.maximum(v, _perm(v, iota ^ d))
    return v


def _splat_f(x):
    return jnp.full((16,), x, dtype=F32)


def _splat_i(x):
    return jnp.full((16,), x, dtype=I32)


def _vlog(x):
    bits = plsc.bitcast(x, I32)
    e = ((bits >> 23) & 0xFF) - 126
    m = plsc.bitcast((bits & 0x007FFFFF) | 0x3F000000, F32)
    small = m < F32(0.7071067811865476)
    m = jnp.where(small, m + m, m)
    e = jnp.where(small, e - 1, e)
    ef = e.astype(F32)
    f = m - F32(1.0)
    z = f * f
    p = F32(7.0376836292e-2)
    for c in (-1.1514610310e-1, 1.1676998740e-1, -1.2420140846e-1,
              1.4249322787e-1, -1.6668057665e-1, 2.0000714765e-1,
              -2.4999993993e-1, 3.3333331174e-1):
        p = p * f + F32(c)
    y = f * z * p
    y = y + ef * F32(-2.12194440e-4)
    y = y - F32(0.5) * z
    return f + y + ef * F32(0.693359375)


def _row_sumexp_8(tref, row, iota):
    idx = _splat_i(row * 8) + (iota & 7)
    v = plsc.load_gather(tref, [idx])
    s = jnp.where(iota < 8, jnp.exp(v), F32(0.0))
    return _allsum(s, iota)


def _scatter1(ref, pos, vec, iota):
    plsc.store_scatter(ref, [_splat_i(pos)], vec, mask=iota == 0)


def _sc_body(c_h, t_h, e_h, p_h, ys_h, out_h,
             ev, tv, pv, cv, ysm, esums, tsums, psums,
             cmem, lzpm, basemem, plvm, etab, totmem, finmem, outmem, shared,
             dsem0, dsem1, dsem2, dsem3, dsem4):
    iota = _iota16()
    wid = lax.axis_index("s")

    cp0 = pltpu.async_copy(c_h, cv, dsem0)
    cp1 = pltpu.async_copy(t_h, tv, dsem1)
    cp2 = pltpu.async_copy(e_h, ev, dsem2)
    cp3 = pltpu.async_copy(p_h, pv, dsem3)
    cp4 = pltpu.async_copy(ys_h, ysm, dsem4)
    cp0.wait()
    cp1.wait()
    cp2.wait()
    cp3.wait()
    cp4.wait()

    cvec = cv[...]
    s_c = _allsum(jnp.where(iota < _I, jnp.exp(cvec), F32(0.0)), iota)
    c_l = cvec - _vlog(s_c)
    cmem[...] = c_l

    one = _splat_f(F32(1.0))
    esums[pl.ds(0, 16)] = one
    esums[pl.ds(16, 16)] = one
    tsums[pl.ds(0, 16)] = one
    tsums[pl.ds(16, 16)] = one
    psums[...] = one
    for r in range(_NROW):
        acc = jnp.exp(ev[pl.ds(r * 128, 16)])
        for k in range(1, 8):
            acc = acc + jnp.exp(ev[pl.ds(r * 128 + 16 * k, 16)])
        _scatter1(esums, r, _allsum(acc, iota), iota)
        _scatter1(tsums, r, _row_sumexp_8(tv, r, iota), iota)
    for i in range(_I):
        _scatter1(psums, i, _row_sumexp_8(pv, i, iota), iota)
    lzpm[...] = _vlog(psums[...])

    plvm[pl.ds(0, 16)] = pv[pl.ds(0, 16)] - plsc.load_gather(lzpm, [iota >> 3])
    plvm[pl.ds(16, 16)] = pv[pl.ds(16, 16)] - plsc.load_gather(lzpm, [_splat_i(2)])

    diag0 = plsc.load_gather(tv, [iota * 8 + (iota & 7)])
    r2 = iota + 16
    didx2 = jnp.minimum(r2 * 8 + (r2 & 7), 191)
    diag1 = plsc.load_gather(tv, [didx2])
    d0 = diag0 - _vlog(tsums[pl.ds(0, 16)])
    d1 = diag1 - _vlog(tsums[pl.ds(16, 16)])
    base0 = plsc.load_gather(cmem, [iota >> 3]) + d0 - _vlog(esums[pl.ds(0, 16)])
    base1 = plsc.load_gather(cmem, [_splat_i(2)]) + d1 - _vlog(esums[pl.ds(16, 16)])
    basemem[pl.ds(0, 16)] = base0
    basemem[pl.ds(16, 16)] = base1

    a_row = wid >> 1
    b_lo = (wid & 1) * 4
    rows = [a_row] + [8 + b_lo + m for m in range(4)] + [16 + n for n in range(8)]

    for tc in range(8):
        yv = ysm[pl.ds(tc * 16, 16)]
        for rpos, row in enumerate(rows):
            bspl = plsc.load_gather(basemem, [_splat_i(row)])
            g = plsc.load_gather(ev, [_splat_i(row * 128) + yv])
            etab[pl.ds((tc * 13 + rpos) * 16, 16)] = bspl + g

    for j in range(104):
        etab[pl.ds(j * 16, 16)] = jnp.exp(etab[pl.ds(j * 16, 16)]) * _SCALE

    pr_rows = [plsc.load_gather(plvm, [_splat_i(row)]) for row in rows]

    lc = _splat_f(_LN2_120)

    def _q(tc, m, n):
        o = tc * 13 * 16
        return (etab[pl.ds(o, 16)]
                + etab[pl.ds(o + (1 + m) * 16, 16)]
                + etab[pl.ds(o + (5 + n) * 16, 16)])

    for m in range(4):
        for n in range(0, 8, 2):
            acc_a = jnp.zeros((16,), dtype=F32)
            acc_b = jnp.zeros((16,), dtype=F32)
            for tp in range(4):
                pa = _q(2 * tp, m, n) * _q(2 * tp + 1, m, n)
                pb = _q(2 * tp, m, n + 1) * _q(2 * tp + 1, m, n + 1)
                acc_a = acc_a + (_vlog(pa) - lc)
                acc_b = acc_b + (_vlog(pb) - lc)
            tot_a = _allsum(acc_a, iota) + pr_rows[0] + pr_rows[1 + m] + pr_rows[5 + n]
            tot_b = _allsum(acc_b, iota) + pr_rows[0] + pr_rows[1 + m] + pr_rows[6 + n]
            plsc.store_scatter(totmem, [_splat_i(m * 8 + n)], tot_a, mask=iota == 0)
            plsc.store_scatter(totmem, [_splat_i(m * 8 + n + 1)], tot_b, mask=iota == 0)

    pltpu.sync_copy(totmem, shared.at[pl.ds(wid * _SPW, _SPW)])
    plsc.subcore_barrier()

    @pl.when(wid == 0)
    def _final():
        pltpu.sync_copy(shared, finmem)
        mv = finmem[pl.ds(0, 16)]
        for b in range(1, 32):
            mv = jnp.maximum(mv, finmem[pl.ds(b * 16, 16)])
        mspl = _allmax(mv, iota)
        sacc = jnp.zeros((16,), dtype=F32)
        for b in range(32):
            sacc = sacc + jnp.exp(finmem[pl.ds(b * 16, 16)] - mspl)
        outmem[...] = mspl + _vlog(_allsum(sacc, iota))
        pltpu.sync_copy(outmem, out_h)


_hmm_sc = functools.partial(
    pl.kernel,
    out_type=jax.ShapeDtypeStruct((16,), F32),
    mesh=plsc.VectorSubcoreMesh(
        core_axis_name="c", subcore_axis_name="s", num_cores=1),
    compiler_params=pltpu.CompilerParams(needs_layout_passes=False),
    scratch_types=[
        pltpu.VMEM((3072,), F32),
        pltpu.VMEM((192,), F32),
        pltpu.VMEM((32,), F32),
        pltpu.VMEM((16,), F32),
        pltpu.VMEM((128,), I32),
        pltpu.VMEM((32,), F32),
        pltpu.VMEM((32,), F32),
        pltpu.VMEM((16,), F32),
        pltpu.VMEM((16,), F32),
        pltpu.VMEM((16,), F32),
        pltpu.VMEM((32,), F32),
        pltpu.VMEM((32,), F32),
        pltpu.VMEM((1664,), F32),
        pltpu.VMEM((32,), F32),
        pltpu.VMEM((512,), F32),
        pltpu.VMEM((16,), F32),
        pltpu.VMEM_SHARED((512,), F32),
        pltpu.SemaphoreType.DMA,
        pltpu.SemaphoreType.DMA,
        pltpu.SemaphoreType.DMA,
        pltpu.SemaphoreType.DMA,
        pltpu.SemaphoreType.DMA,
    ],
)(_sc_body)


def kernel(choice, transition, emission, prior, ys):
    c_pad = jnp.zeros((16,), F32).at[:_I].set(choice.astype(F32))
    t_flat = transition.astype(F32).reshape(-1)
    e_flat = emission.astype(F32).reshape(-1)
    p_pad = jnp.zeros((32,), F32).at[:_I * _S].set(prior.astype(F32).reshape(-1))
    ys32 = ys.astype(I32)
    out = _hmm_sc(c_pad, t_flat, e_flat, p_pad, ys32)
    return out[0]


# speedup vs baseline: 1.0670x; 1.0670x over previous
import functools

import numpy as np

import jax
import jax.numpy as jnp
from jax import lax
from jax.experimental import pallas as pl
from jax.experimental.pallas import tpu as pltpu
from jax.experimental.pallas import tpu_sc as plsc

F32 = np.float32
I32 = np.int32

_I = 3
_S = 8
_A = 128
_T = 128
_NSUB = 16
_SPW = 32
_NROW = 24

_SCALE = F32(2.0 ** 60)
_LN2_120 = F32(120 * 0.6931471805599453)


def _iota16():
    return lax.iota(I32, 16)


def _perm(v, idx):
    return v.at[idx].get(mode="promise_in_bounds")


def _allsum(v, iota):
    for d in (1, 2, 4, 8):
        v = v + _perm(v, iota ^ d)
    return v


def _allmax(v, iota):
    for d in (1, 2, 4, 8):
        v = jnp.maximum(v, _perm(v, iota ^ d))
    return v


def _splat_f(x):
    return jnp.full((16,), x, dtype=F32)


def _splat_i(x):
    return jnp.full((16,), x, dtype=I32)


def _vlog(x):
    bits = plsc.bitcast(x, I32)
    e = ((bits >> 23) & 0xFF) - 126
    m = plsc.bitcast((bits & 0x007FFFFF) | 0x3F000000, F32)
    small = m < F32(0.7071067811865476)
    m = jnp.where(small, m + m, m)
    e = jnp.where(small, e - 1, e)
    ef = e.astype(F32)
    f = m - F32(1.0)
    z = f * f
    p = F32(7.0376836292e-2)
    for c in (-1.1514610310e-1, 1.1676998740e-1, -1.2420140846e-1,
              1.4249322787e-1, -1.6668057665e-1, 2.0000714765e-1,
              -2.4999993993e-1, 3.3333331174e-1):
        p = p * f + F32(c)
    y = f * z * p
    y = y + ef * F32(-2.12194440e-4)
    y = y - F32(0.5) * z
    return f + y + ef * F32(0.693359375)


def _row_sumexp_8(tref, row, iota):
    idx = _splat_i(row * 8) + (iota & 7)
    v = plsc.load_gather(tref, [idx])
    s = jnp.where(iota < 8, jnp.exp(v), F32(0.0))
    return _allsum(s, iota)


def _scatter1(ref, pos, vec, iota):
    plsc.store_scatter(ref, [_splat_i(pos)], vec, mask=iota == 0)


def _sc_body(c_h, t_h, e_h, p_h, ys_h, out_h,
             ev, tv, pv, cv, ysm, esums, tsums, psums,
             cmem, lzpm, basemem, plvm, etab, totmem, finmem, outmem, shared,
             dsem0, dsem1, dsem2, dsem3, dsem4):
    iota = _iota16()
    wid = lax.axis_index("s")

    cp0 = pltpu.async_copy(c_h, cv, dsem0)
    cp1 = pltpu.async_copy(t_h, tv, dsem1)
    cp2 = pltpu.async_copy(e_h, ev, dsem2)
    cp3 = pltpu.async_copy(p_h, pv, dsem3)
    cp4 = pltpu.async_copy(ys_h, ysm, dsem4)
    cp0.wait()
    cp1.wait()
    cp2.wait()
    cp3.wait()
    cp4.wait()

    cvec = cv[...]
    s_c = _allsum(jnp.where(iota < _I, jnp.exp(cvec), F32(0.0)), iota)
    c_l = cvec - _vlog(s_c)
    cmem[...] = c_l

    one = _splat_f(F32(1.0))
    esums[pl.ds(0, 16)] = one
    esums[pl.ds(16, 16)] = one
    tsums[pl.ds(0, 16)] = one
    tsums[pl.ds(16, 16)] = one
    psums[...] = one
    for r in range(0, _NROW, 2):
        acc0 = jnp.exp(ev[pl.ds(r * 128, 16)])
        acc1 = jnp.exp(ev[pl.ds((r + 1) * 128, 16)])
        for k in range(1, 8):
            acc0 = acc0 + jnp.exp(ev[pl.ds(r * 128 + 16 * k, 16)])
            acc1 = acc1 + jnp.exp(ev[pl.ds((r + 1) * 128 + 16 * k, 16)])
        _scatter1(esums, r, _allsum(acc0, iota), iota)
        _scatter1(esums, r + 1, _allsum(acc1, iota), iota)
        _scatter1(tsums, r, _row_sumexp_8(tv, r, iota), iota)
        _scatter1(tsums, r + 1, _row_sumexp_8(tv, r + 1, iota), iota)
    for i in range(_I):
        _scatter1(psums, i, _row_sumexp_8(pv, i, iota), iota)
    lzpm[...] = _vlog(psums[...])

    plvm[pl.ds(0, 16)] = pv[pl.ds(0, 16)] - plsc.load_gather(lzpm, [iota >> 3])
    plvm[pl.ds(16, 16)] = pv[pl.ds(16, 16)] - plsc.load_gather(lzpm, [_splat_i(2)])

    diag0 = plsc.load_gather(tv, [iota * 8 + (iota & 7)])
    r2 = iota + 16
    didx2 = jnp.minimum(r2 * 8 + (r2 & 7), 191)
    diag1 = plsc.load_gather(tv, [didx2])
    d0 = diag0 - _vlog(tsums[pl.ds(0, 16)])
    d1 = diag1 - _vlog(tsums[pl.ds(16, 16)])
    base0 = plsc.load_gather(cmem, [iota >> 3]) + d0 - _vlog(esums[pl.ds(0, 16)])
    base1 = plsc.load_gather(cmem, [_splat_i(2)]) + d1 - _vlog(esums[pl.ds(16, 16)])
    basemem[pl.ds(0, 16)] = base0
    basemem[pl.ds(16, 16)] = base1

    a_row = wid >> 1
    b_lo = (wid & 1) * 4
    rows = [a_row] + [8 + b_lo + m for m in range(4)] + [16 + n for n in range(8)]

    for tc in range(8):
        yv = ysm[pl.ds(tc * 16, 16)]
        for rpos, row in enumerate(rows):
            bspl = plsc.load_gather(basemem, [_splat_i(row)])
            g = plsc.load_gather(ev, [_splat_i(row * 128) + yv])
            etab[pl.ds((tc * 13 + rpos) * 16, 16)] = bspl + g

    for j in range(104):
        etab[pl.ds(j * 16, 16)] = jnp.exp(etab[pl.ds(j * 16, 16)]) * _SCALE

    pr_rows = [plsc.load_gather(plvm, [_splat_i(row)]) for row in rows]

    lc = _splat_f(_LN2_120)

    def _q(tc, m, n):
        o = tc * 13 * 16
        return (etab[pl.ds(o, 16)]
                + etab[pl.ds(o + (1 + m) * 16, 16)]
                + etab[pl.ds(o + (5 + n) * 16, 16)])

    for m in range(4):
        for n in range(0, 8, 4):
            accs = [jnp.zeros((16,), dtype=F32) for _ in range(4)]
            for tp in range(4):
                ps = [_q(2 * tp, m, n + i) * _q(2 * tp + 1, m, n + i)
                      for i in range(4)]
                accs = [a + (_vlog(p) - lc) for a, p in zip(accs, ps)]
            for i in range(4):
                tot = (_allsum(accs[i], iota)
                       + pr_rows[0] + pr_rows[1 + m] + pr_rows[5 + n + i])
                plsc.store_scatter(totmem, [_splat_i(m * 8 + n + i)], tot,
                                   mask=iota == 0)

    pltpu.sync_copy(totmem, shared.at[pl.ds(wid * _SPW, _SPW)])
    plsc.subcore_barrier()

    @pl.when(wid == 0)
    def _final():
        pltpu.sync_copy(shared, finmem)
        mv = finmem[pl.ds(0, 16)]
        for b in range(1, 32):
            mv = jnp.maximum(mv, finmem[pl.ds(b * 16, 16)])
        mspl = _allmax(mv, iota)
        sacc = jnp.zeros((16,), dtype=F32)
        for b in range(32):
            sacc = sacc + jnp.exp(finmem[pl.ds(b * 16, 16)] - mspl)
        outmem[...] = mspl + _vlog(_allsum(sacc, iota))
        pltpu.sync_copy(outmem, out_h)


_hmm_sc = functools.partial(
    pl.kernel,
    out_type=jax.ShapeDtypeStruct((16,), F32),
    mesh=plsc.VectorSubcoreMesh(
        core_axis_name="c", subcore_axis_name="s", num_cores=1),
    compiler_params=pltpu.CompilerParams(needs_layout_passes=False),
    scratch_types=[
        pltpu.VMEM((3072,), F32),
        pltpu.VMEM((192,), F32),
        pltpu.VMEM((32,), F32),
        pltpu.VMEM((16,), F32),
        pltpu.VMEM((128,), I32),
        pltpu.VMEM((32,), F32),
        pltpu.VMEM((32,), F32),
        pltpu.VMEM((16,), F32),
        pltpu.VMEM((16,), F32),
        pltpu.VMEM((16,), F32),
        pltpu.VMEM((32,), F32),
        pltpu.VMEM((32,), F32),
        pltpu.VMEM((1664,), F32),
        pltpu.VMEM((32,), F32),
        pltpu.VMEM((512,), F32),
        pltpu.VMEM((16,), F32),
        pltpu.VMEM_SHARED((512,), F32),
        pltpu.SemaphoreType.DMA,
        pltpu.SemaphoreType.DMA,
        pltpu.SemaphoreType.DMA,
        pltpu.SemaphoreType.DMA,
        pltpu.SemaphoreType.DMA,
    ],
)(_sc_body)


def kernel(choice, transition, emission, prior, ys):
    c_pad = jnp.zeros((16,), F32).at[:_I].set(choice.astype(F32))
    t_flat = transition.astype(F32).reshape(-1)
    e_flat = emission.astype(F32).reshape(-1)
    p_pad = jnp.zeros((32,), F32).at[:_I * _S].set(prior.astype(F32).reshape(-1))
    ys32 = ys.astype(I32)
    out = _hmm_sc(c_pad, t_flat, e_flat, p_pad, ys32)
    return out[0]


# speedup vs baseline: 1.1156x; 1.0456x over previous
import functools

import numpy as np

import jax
import jax.numpy as jnp
from jax import lax
from jax.experimental import pallas as pl
from jax.experimental.pallas import tpu as pltpu
from jax.experimental.pallas import tpu_sc as plsc

F32 = np.float32
I32 = np.int32

_I = 3
_S = 8
_A = 128
_T = 128
_NSUB = 16
_SPW = 32
_NROW = 24

_SCALE = F32(2.0 ** 60)
_LN2_120 = F32(120 * 0.6931471805599453)


def _iota16():
    return lax.iota(I32, 16)


def _perm(v, idx):
    return v.at[idx].get(mode="promise_in_bounds")


def _allsum(v, iota):
    for d in (1, 2, 4, 8):
        v = v + _perm(v, iota ^ d)
    return v


def _allmax(v, iota):
    for d in (1, 2, 4, 8):
        v = jnp.maximum(v, _perm(v, iota ^ d))
    return v


def _splat_f(x):
    return jnp.full((16,), x, dtype=F32)


def _splat_i(x):
    return jnp.full((16,), x, dtype=I32)


def _vlog(x):
    bits = plsc.bitcast(x, I32)
    e = ((bits >> 23) & 0xFF) - 126
    m = plsc.bitcast((bits & 0x007FFFFF) | 0x3F000000, F32)
    small = m < F32(0.7071067811865476)
    m = jnp.where(small, m + m, m)
    e = jnp.where(small, e - 1, e)
    ef = e.astype(F32)
    f = m - F32(1.0)
    z = f * f
    p = F32(7.0376836292e-2)
    for c in (-1.1514610310e-1, 1.1676998740e-1, -1.2420140846e-1,
              1.4249322787e-1, -1.6668057665e-1, 2.0000714765e-1,
              -2.4999993993e-1, 3.3333331174e-1):
        p = p * f + F32(c)
    y = f * z * p
    y = y + ef * F32(-2.12194440e-4)
    y = y - F32(0.5) * z
    return f + y + ef * F32(0.693359375)


def _row_sumexp_8(tref, row, iota):
    idx = _splat_i(row * 8) + (iota & 7)
    v = plsc.load_gather(tref, [idx])
    s = jnp.where(iota < 8, jnp.exp(v), F32(0.0))
    return _allsum(s, iota)


def _scatter1(ref, pos, vec, iota):
    plsc.store_scatter(ref, [_splat_i(pos)], vec, mask=iota == 0)


def _sc_body(c_h, t_h, e_h, p_h, ys_h, out_h,
             ev, tv, pv, cv, ysm, esums, tsums, psums,
             cmem, lzpm, basemem, plvm, etab, totmem, finmem, outmem, shared,
             dsem0, dsem1, dsem2, dsem3, dsem4):
    iota = _iota16()
    wid = lax.axis_index("s")

    cp0 = pltpu.async_copy(c_h, cv, dsem0)
    cp1 = pltpu.async_copy(t_h, tv, dsem1)
    cp2 = pltpu.async_copy(e_h, ev, dsem2)
    cp3 = pltpu.async_copy(p_h, pv, dsem3)
    cp4 = pltpu.async_copy(ys_h, ysm, dsem4)
    cp0.wait()
    cp1.wait()
    cp2.wait()
    cp3.wait()
    cp4.wait()

    cvec = cv[...]
    s_c = _allsum(jnp.where(iota < _I, jnp.exp(cvec), F32(0.0)), iota)
    c_l = cvec - _vlog(s_c)
    cmem[...] = c_l

    one = _splat_f(F32(1.0))
    esums[pl.ds(0, 16)] = one
    esums[pl.ds(16, 16)] = one
    for r in range(0, _NROW, 2):
        acc0 = jnp.exp(ev[pl.ds(r * 128, 16)])
        acc1 = jnp.exp(ev[pl.ds((r + 1) * 128, 16)])
        for k in range(1, 8):
            acc0 = acc0 + jnp.exp(ev[pl.ds(r * 128 + 16 * k, 16)])
            acc1 = acc1 + jnp.exp(ev[pl.ds((r + 1) * 128 + 16 * k, 16)])
        _scatter1(esums, r, _allsum(acc0, iota), iota)
        _scatter1(esums, r + 1, _allsum(acc1, iota), iota)

    ts0 = jnp.zeros((16,), dtype=F32)
    ts1 = jnp.zeros((16,), dtype=F32)
    psv = jnp.zeros((16,), dtype=F32)
    for k in range(8):
        ts0 = ts0 + jnp.exp(plsc.load_gather(tv, [iota * 8 + k]))
        idx1 = jnp.minimum((iota + 16) * 8 + k, 191)
        ts1 = ts1 + jnp.exp(plsc.load_gather(tv, [idx1]))
        psv = psv + jnp.exp(plsc.load_gather(pv, [jnp.minimum(iota * 8 + k, 31)]))
    tsums[pl.ds(0, 16)] = ts0
    tsums[pl.ds(16, 16)] = ts1
    lzpm[...] = _vlog(psv)

    plvm[pl.ds(0, 16)] = pv[pl.ds(0, 16)] - plsc.load_gather(lzpm, [iota >> 3])
    plvm[pl.ds(16, 16)] = pv[pl.ds(16, 16)] - plsc.load_gather(lzpm, [_splat_i(2)])

    diag0 = plsc.load_gather(tv, [iota * 8 + (iota & 7)])
    r2 = iota + 16
    didx2 = jnp.minimum(r2 * 8 + (r2 & 7), 191)
    diag1 = plsc.load_gather(tv, [didx2])
    d0 = diag0 - _vlog(tsums[pl.ds(0, 16)])
    d1 = diag1 - _vlog(tsums[pl.ds(16, 16)])
    base0 = plsc.load_gather(cmem, [iota >> 3]) + d0 - _vlog(esums[pl.ds(0, 16)])
    base1 = plsc.load_gather(cmem, [_splat_i(2)]) + d1 - _vlog(esums[pl.ds(16, 16)])
    basemem[pl.ds(0, 16)] = base0
    basemem[pl.ds(16, 16)] = base1

    a_row = wid >> 1
    b_lo = (wid & 1) * 4
    rows = [a_row] + [8 + b_lo + m for m in range(4)] + [16 + n for n in range(8)]

    for tc in range(8):
        yv = ysm[pl.ds(tc * 16, 16)]
        for rpos, row in enumerate(rows):
            bspl = plsc.load_gather(basemem, [_splat_i(row)])
            g = plsc.load_gather(ev, [_splat_i(row * 128) + yv])
            etab[pl.ds((tc * 13 + rpos) * 16, 16)] = bspl + g

    for j in range(104):
        etab[pl.ds(j * 16, 16)] = jnp.exp(etab[pl.ds(j * 16, 16)]) * _SCALE

    pr_rows = [plsc.load_gather(plvm, [_splat_i(row)]) for row in rows]

    lc = _splat_f(_LN2_120)

    def _q(tc, m, n):
        o = tc * 13 * 16
        return (etab[pl.ds(o, 16)]
                + etab[pl.ds(o + (1 + m) * 16, 16)]
                + etab[pl.ds(o + (5 + n) * 16, 16)])

    for m in range(4):
        accs = [jnp.zeros((16,), dtype=F32) for _ in range(8)]
        for tp in range(4):
            ps = [_q(2 * tp, m, i) * _q(2 * tp + 1, m, i) for i in range(8)]
            accs = [a + (_vlog(p) - lc) for a, p in zip(accs, ps)]
        for i in range(8):
            tot = (_allsum(accs[i], iota)
                   + pr_rows[0] + pr_rows[1 + m] + pr_rows[5 + i])
            plsc.store_scatter(totmem, [_splat_i(m * 8 + i)], tot,
                               mask=iota == 0)

    pltpu.sync_copy(totmem, shared.at[pl.ds(wid * _SPW, _SPW)])
    plsc.subcore_barrier()

    @pl.when(wid == 0)
    def _final():
        pltpu.sync_copy(shared, finmem)
        mv = finmem[pl.ds(0, 16)]
        for b in range(1, 32):
            mv = jnp.maximum(mv, finmem[pl.ds(b * 16, 16)])
        mspl = _allmax(mv, iota)
        sacc = jnp.zeros((16,), dtype=F32)
        for b in range(32):
            sacc = sacc + jnp.exp(finmem[pl.ds(b * 16, 16)] - mspl)
        outmem[...] = mspl + _vlog(_allsum(sacc, iota))
        pltpu.sync_copy(outmem, out_h)


_hmm_sc = functools.partial(
    pl.kernel,
    out_type=jax.ShapeDtypeStruct((16,), F32),
    mesh=plsc.VectorSubcoreMesh(
        core_axis_name="c", subcore_axis_name="s", num_cores=1),
    compiler_params=pltpu.CompilerParams(needs_layout_passes=False),
    scratch_types=[
        pltpu.VMEM((3072,), F32),
        pltpu.VMEM((192,), F32),
        pltpu.VMEM((32,), F32),
        pltpu.VMEM((16,), F32),
        pltpu.VMEM((128,), I32),
        pltpu.VMEM((32,), F32),
        pltpu.VMEM((32,), F32),
        pltpu.VMEM((16,), F32),
        pltpu.VMEM((16,), F32),
        pltpu.VMEM((16,), F32),
        pltpu.VMEM((32,), F32),
        pltpu.VMEM((32,), F32),
        pltpu.VMEM((1664,), F32),
        pltpu.VMEM((32,), F32),
        pltpu.VMEM((512,), F32),
        pltpu.VMEM((16,), F32),
        pltpu.VMEM_SHARED((512,), F32),
        pltpu.SemaphoreType.DMA,
        pltpu.SemaphoreType.DMA,
        pltpu.SemaphoreType.DMA,
        pltpu.SemaphoreType.DMA,
        pltpu.SemaphoreType.DMA,
    ],
)(_sc_body)


def kernel(choice, transition, emission, prior, ys):
    c_pad = jnp.zeros((16,), F32).at[:_I].set(choice.astype(F32))
    t_flat = transition.astype(F32).reshape(-1)
    e_flat = emission.astype(F32).reshape(-1)
    p_pad = jnp.zeros((32,), F32).at[:_I * _S].set(prior.astype(F32).reshape(-1))
    ys32 = ys.astype(I32)
    out = _hmm_sc(c_pad, t_flat, e_flat, p_pad, ys32)
    return out[0]


# speedup vs baseline: 1.2212x; 1.0946x over previous
import functools

import numpy as np

import jax
import jax.numpy as jnp
from jax import lax
from jax.experimental import pallas as pl
from jax.experimental.pallas import tpu as pltpu
from jax.experimental.pallas import tpu_sc as plsc

F32 = np.float32
I32 = np.int32

_I = 3
_S = 8
_A = 128
_T = 128
_NSUB = 16
_SPW = 32
_NROW = 24

_SCALE = F32(2.0 ** 60)
_LN2_120 = F32(120 * 0.6931471805599453)


def _iota16():
    return lax.iota(I32, 16)


def _perm(v, idx):
    return v.at[idx].get(mode="promise_in_bounds")


def _allsum(v, iota):
    for d in (1, 2, 4, 8):
        v = v + _perm(v, iota ^ d)
    return v


def _allmax(v, iota):
    for d in (1, 2, 4, 8):
        v = jnp.maximum(v, _perm(v, iota ^ d))
    return v


def _splat_f(x):
    return jnp.full((16,), x, dtype=F32)


def _splat_i(x):
    return jnp.full((16,), x, dtype=I32)


_LOGP = (0.9999995231628418, -0.49996355175971985, 0.33265241980552673,
         -0.24453333020210266, 0.17659756541252136, -0.10679824650287628,
         0.04365880414843559, -0.008466342464089394)


def _vlog(x):
    bits = plsc.bitcast(x, I32)
    e = ((bits >> 23) & 0xFF) - 127
    m = plsc.bitcast((bits & 0x007FFFFF) | 0x3F800000, F32)
    ef = e.astype(F32)
    f = m - F32(1.0)
    p = F32(_LOGP[-1])
    for c in _LOGP[-2::-1]:
        p = p * f + F32(c)
    return f * p + (ef * F32(0.693359375) + ef * F32(-2.12194440e-4))


def _row_sumexp_8(tref, row, iota):
    idx = _splat_i(row * 8) + (iota & 7)
    v = plsc.load_gather(tref, [idx])
    s = jnp.where(iota < 8, jnp.exp(v), F32(0.0))
    return _allsum(s, iota)


def _scatter1(ref, pos, vec, iota):
    plsc.store_scatter(ref, [_splat_i(pos)], vec, mask=iota == 0)


def _sc_body(c_h, t_h, e_h, p_h, ys_h, out_h,
             ev, tv, pv, cv, ysm, esums, tsums, psums,
             cmem, lzpm, basemem, plvm, etab, totmem, finmem, outmem, shared,
             dsem0, dsem1, dsem2, dsem3, dsem4):
    iota = _iota16()
    wid = lax.axis_index("s")

    cp0 = pltpu.async_copy(c_h, cv, dsem0)
    cp1 = pltpu.async_copy(t_h, tv, dsem1)
    cp2 = pltpu.async_copy(e_h, ev, dsem2)
    cp3 = pltpu.async_copy(p_h, pv, dsem3)
    cp4 = pltpu.async_copy(ys_h, ysm, dsem4)
    cp0.wait()
    cp1.wait()
    cp2.wait()
    cp3.wait()
    cp4.wait()

    cvec = cv[...]
    s_c = _allsum(jnp.where(iota < _I, jnp.exp(cvec), F32(0.0)), iota)
    c_l = cvec - _vlog(s_c)
    cmem[...] = c_l

    one = _splat_f(F32(1.0))
    esums[pl.ds(0, 16)] = one
    esums[pl.ds(16, 16)] = one
    for r in range(0, _NROW, 4):
        accs = [jnp.exp(ev[pl.ds((r + j) * 128, 16)]) for j in range(4)]
        for k in range(1, 8):
            accs = [a + jnp.exp(ev[pl.ds((r + j) * 128 + 16 * k, 16)])
                    for j, a in enumerate(accs)]
        for j in range(4):
            _scatter1(esums, r + j, _allsum(accs[j], iota), iota)

    ts0 = jnp.zeros((16,), dtype=F32)
    ts1 = jnp.zeros((16,), dtype=F32)
    psv = jnp.zeros((16,), dtype=F32)
    for k in range(8):
        ts0 = ts0 + jnp.exp(plsc.load_gather(tv, [iota * 8 + k]))
        idx1 = jnp.minimum((iota + 16) * 8 + k, 191)
        ts1 = ts1 + jnp.exp(plsc.load_gather(tv, [idx1]))
        psv = psv + jnp.exp(plsc.load_gather(pv, [jnp.minimum(iota * 8 + k, 31)]))
    tsums[pl.ds(0, 16)] = ts0
    tsums[pl.ds(16, 16)] = ts1
    lzpm[...] = _vlog(psv)

    plvm[pl.ds(0, 16)] = pv[pl.ds(0, 16)] - plsc.load_gather(lzpm, [iota >> 3])
    plvm[pl.ds(16, 16)] = pv[pl.ds(16, 16)] - plsc.load_gather(lzpm, [_splat_i(2)])

    diag0 = plsc.load_gather(tv, [iota * 8 + (iota & 7)])
    r2 = iota + 16
    didx2 = jnp.minimum(r2 * 8 + (r2 & 7), 191)
    diag1 = plsc.load_gather(tv, [didx2])
    d0 = diag0 - _vlog(tsums[pl.ds(0, 16)])
    d1 = diag1 - _vlog(tsums[pl.ds(16, 16)])
    base0 = plsc.load_gather(cmem, [iota >> 3]) + d0 - _vlog(esums[pl.ds(0, 16)])
    base1 = plsc.load_gather(cmem, [_splat_i(2)]) + d1 - _vlog(esums[pl.ds(16, 16)])
    basemem[pl.ds(0, 16)] = base0
    basemem[pl.ds(16, 16)] = base1

    a_row = wid >> 1
    b_lo = (wid & 1) * 4
    rows = [a_row] + [8 + b_lo + m for m in range(4)] + [16 + n for n in range(8)]

    for tc in range(8):
        yv = ysm[pl.ds(tc * 16, 16)]
        for rpos, row in enumerate(rows):
            bspl = plsc.load_gather(basemem, [_splat_i(row)])
            g = plsc.load_gather(ev, [_splat_i(row * 128) + yv])
            etab[pl.ds((tc * 13 + rpos) * 16, 16)] = bspl + g

    for j in range(104):
        etab[pl.ds(j * 16, 16)] = jnp.exp(etab[pl.ds(j * 16, 16)]) * _SCALE

    pr_rows = [plsc.load_gather(plvm, [_splat_i(row)]) for row in rows]

    lc = _splat_f(_LN2_120)

    def _q(tc, m, n):
        o = tc * 13 * 16
        return (etab[pl.ds(o, 16)]
                + etab[pl.ds(o + (1 + m) * 16, 16)]
                + etab[pl.ds(o + (5 + n) * 16, 16)])

    for m in range(4):
        accs = [jnp.zeros((16,), dtype=F32) for _ in range(8)]
        for tp in range(4):
            ps = [_q(2 * tp, m, i) * _q(2 * tp + 1, m, i) for i in range(8)]
            accs = [a + (_vlog(p) - lc) for a, p in zip(accs, ps)]
        for i in range(8):
            tot = (_allsum(accs[i], iota)
                   + pr_rows[0] + pr_rows[1 + m] + pr_rows[5 + i])
            plsc.store_scatter(totmem, [_splat_i(m * 8 + i)], tot,
                               mask=iota == 0)

    pltpu.sync_copy(totmem, shared.at[pl.ds(wid * _SPW, _SPW)])
    plsc.subcore_barrier()

    @pl.when(wid == 0)
    def _final():
        pltpu.sync_copy(shared, finmem)
        mv = finmem[pl.ds(0, 16)]
        for b in range(1, 32):
            mv = jnp.maximum(mv, finmem[pl.ds(b * 16, 16)])
        mspl = _allmax(mv, iota)
        sacc = jnp.zeros((16,), dtype=F32)
        for b in range(32):
            sacc = sacc + jnp.exp(finmem[pl.ds(b * 16, 16)] - mspl)
        outmem[...] = mspl + _vlog(_allsum(sacc, iota))
        pltpu.sync_copy(outmem, out_h)


_hmm_sc = functools.partial(
    pl.kernel,
    out_type=jax.ShapeDtypeStruct((16,), F32),
    mesh=plsc.VectorSubcoreMesh(
        core_axis_name="c", subcore_axis_name="s", num_cores=1),
    compiler_params=pltpu.CompilerParams(needs_layout_passes=False),
    scratch_types=[
        pltpu.VMEM((3072,), F32),
        pltpu.VMEM((192,), F32),
        pltpu.VMEM((32,), F32),
        pltpu.VMEM((16,), F32),
        pltpu.VMEM((128,), I32),
        pltpu.VMEM((32,), F32),
        pltpu.VMEM((32,), F32),
        pltpu.VMEM((16,), F32),
        pltpu.VMEM((16,), F32),
        pltpu.VMEM((16,), F32),
        pltpu.VMEM((32,), F32),
        pltpu.VMEM((32,), F32),
        pltpu.VMEM((1664,), F32),
        pltpu.VMEM((32,), F32),
        pltpu.VMEM((512,), F32),
        pltpu.VMEM((16,), F32),
        pltpu.VMEM_SHARED((512,), F32),
        pltpu.SemaphoreType.DMA,
        pltpu.SemaphoreType.DMA,
        pltpu.SemaphoreType.DMA,
        pltpu.SemaphoreType.DMA,
        pltpu.SemaphoreType.DMA,
    ],
)(_sc_body)


def kernel(choice, transition, emission, prior, ys):
    c_pad = jnp.zeros((16,), F32).at[:_I].set(choice.astype(F32))
    t_flat = transition.astype(F32).reshape(-1)
    e_flat = emission.astype(F32).reshape(-1)
    p_pad = jnp.zeros((32,), F32).at[:_I * _S].set(prior.astype(F32).reshape(-1))
    ys32 = ys.astype(I32)
    out = _hmm_sc(c_pad, t_flat, e_flat, p_pad, ys32)
    return out[0]


# speedup vs baseline: 1.2404x; 1.0158x over previous
import functools

import numpy as np

import jax
import jax.numpy as jnp
from jax import lax
from jax.experimental import pallas as pl
from jax.experimental.pallas import tpu as pltpu
from jax.experimental.pallas import tpu_sc as plsc

F32 = np.float32
I32 = np.int32

_I = 3
_S = 8
_A = 128
_T = 128
_NSUB = 16
_SPW = 32
_NROW = 24

_SCALE = F32(2.0 ** 60)
_LN2_120 = F32(120 * 0.6931471805599453)


def _iota16():
    return lax.iota(I32, 16)


def _perm(v, idx):
    return v.at[idx].get(mode="promise_in_bounds")


def _allsum(v, iota):
    for d in (1, 2, 4, 8):
        v = v + _perm(v, iota ^ d)
    return v


def _allmax(v, iota):
    for d in (1, 2, 4, 8):
        v = jnp.maximum(v, _perm(v, iota ^ d))
    return v


def _splat_f(x):
    return jnp.full((16,), x, dtype=F32)


def _splat_i(x):
    return jnp.full((16,), x, dtype=I32)


_LOGP = (0.9999995231628418, -0.49996355175971985, 0.33265241980552673,
         -0.24453333020210266, 0.17659756541252136, -0.10679824650287628,
         0.04365880414843559, -0.008466342464089394)


def _vlog(x):
    bits = plsc.bitcast(x, I32)
    e = ((bits >> 23) & 0xFF) - 127
    m = plsc.bitcast((bits & 0x007FFFFF) | 0x3F800000, F32)
    ef = e.astype(F32)
    f = m - F32(1.0)
    p = F32(_LOGP[-1])
    for c in _LOGP[-2::-1]:
        p = p * f + F32(c)
    return f * p + (ef * F32(0.693359375) + ef * F32(-2.12194440e-4))


def _row_sumexp_8(tref, row, iota):
    idx = _splat_i(row * 8) + (iota & 7)
    v = plsc.load_gather(tref, [idx])
    s = jnp.where(iota < 8, jnp.exp(v), F32(0.0))
    return _allsum(s, iota)


def _scatter1(ref, pos, vec, iota):
    plsc.store_scatter(ref, [_splat_i(pos)], vec, mask=iota == 0)


def _sc_body(c_h, t_h, e_h, p_h, ys_h, out_h,
             ev, tv, pv, cv, ysm, esums, tsums, psums,
             cmem, lzpm, basemem, plvm, etab, totmem, finmem, outmem, shared,
             dsem0, dsem1, dsem2, dsem3, dsem4):
    iota = _iota16()
    wid = lax.axis_index("s")

    cp0 = pltpu.async_copy(c_h, cv, dsem0)
    cp1 = pltpu.async_copy(t_h, tv, dsem1)
    cp2 = pltpu.async_copy(e_h, ev, dsem2)
    cp3 = pltpu.async_copy(p_h, pv, dsem3)
    cp4 = pltpu.async_copy(ys_h, ysm, dsem4)
    cp0.wait()
    cp1.wait()
    cp2.wait()
    cp3.wait()
    cp4.wait()

    cvec = cv[...]
    s_c = _allsum(jnp.where(iota < _I, jnp.exp(cvec), F32(0.0)), iota)
    c_l = cvec - _vlog(s_c)
    cmem[...] = c_l

    one = _splat_f(F32(1.0))
    esums[pl.ds(0, 16)] = one
    esums[pl.ds(16, 16)] = one
    for r in range(0, _NROW, 4):
        accs = [jnp.exp(ev[pl.ds((r + j) * 128, 16)]) for j in range(4)]
        for k in range(1, 8):
            accs = [a + jnp.exp(ev[pl.ds((r + j) * 128 + 16 * k, 16)])
                    for j, a in enumerate(accs)]
        for j in range(4):
            _scatter1(esums, r + j, _allsum(accs[j], iota), iota)

    ts0 = jnp.zeros((16,), dtype=F32)
    ts1 = jnp.zeros((16,), dtype=F32)
    psv = jnp.zeros((16,), dtype=F32)
    for k in range(8):
        ts0 = ts0 + jnp.exp(plsc.load_gather(tv, [iota * 8 + k]))
        idx1 = jnp.minimum((iota + 16) * 8 + k, 191)
        ts1 = ts1 + jnp.exp(plsc.load_gather(tv, [idx1]))
        psv = psv + jnp.exp(plsc.load_gather(pv, [jnp.minimum(iota * 8 + k, 31)]))
    tsums[pl.ds(0, 16)] = ts0
    tsums[pl.ds(16, 16)] = ts1
    lzpm[...] = _vlog(psv)

    plvm[pl.ds(0, 16)] = pv[pl.ds(0, 16)] - plsc.load_gather(lzpm, [iota >> 3])
    plvm[pl.ds(16, 16)] = pv[pl.ds(16, 16)] - plsc.load_gather(lzpm, [_splat_i(2)])

    diag0 = plsc.load_gather(tv, [iota * 8 + (iota & 7)])
    r2 = iota + 16
    didx2 = jnp.minimum(r2 * 8 + (r2 & 7), 191)
    diag1 = plsc.load_gather(tv, [didx2])
    d0 = diag0 - _vlog(tsums[pl.ds(0, 16)])
    d1 = diag1 - _vlog(tsums[pl.ds(16, 16)])
    sc60 = _splat_f(F32(60 * 0.6931471805599453))
    base0 = plsc.load_gather(cmem, [iota >> 3]) + d0 - _vlog(esums[pl.ds(0, 16)])
    base1 = plsc.load_gather(cmem, [_splat_i(2)]) + d1 - _vlog(esums[pl.ds(16, 16)])
    basemem[pl.ds(0, 16)] = base0 + sc60
    basemem[pl.ds(16, 16)] = base1 + sc60

    a_row = wid >> 1
    b_lo = (wid & 1) * 4
    rows = [a_row] + [8 + b_lo + m for m in range(4)] + [16 + n for n in range(8)]

    bspl_rows = [plsc.load_gather(basemem, [_splat_i(row)]) for row in rows]
    ridx_rows = [_splat_i(row * 128) for row in rows]
    for tc in range(8):
        yv = ysm[pl.ds(tc * 16, 16)]
        for rpos in range(13):
            g = plsc.load_gather(ev, [ridx_rows[rpos] + yv])
            etab[pl.ds((tc * 13 + rpos) * 16, 16)] = bspl_rows[rpos] + g

    for j in range(104):
        etab[pl.ds(j * 16, 16)] = jnp.exp(etab[pl.ds(j * 16, 16)])

    pr_rows = [plsc.load_gather(plvm, [_splat_i(row)]) for row in rows]

    lc = _splat_f(_LN2_120)

    for m in range(4):
        accs = [jnp.zeros((16,), dtype=F32) for _ in range(8)]
        for tp in range(4):
            o1 = (2 * tp) * 13 * 16
            o2 = (2 * tp + 1) * 13 * 16
            s1 = etab[pl.ds(o1, 16)] + etab[pl.ds(o1 + (1 + m) * 16, 16)]
            s2 = etab[pl.ds(o2, 16)] + etab[pl.ds(o2 + (1 + m) * 16, 16)]
            ps = [(s1 + etab[pl.ds(o1 + (5 + i) * 16, 16)])
                  * (s2 + etab[pl.ds(o2 + (5 + i) * 16, 16)])
                  for i in range(8)]
            accs = [a + (_vlog(p) - lc) for a, p in zip(accs, ps)]
        for i in range(8):
            tot = (_allsum(accs[i], iota)
                   + pr_rows[0] + pr_rows[1 + m] + pr_rows[5 + i])
            plsc.store_scatter(totmem, [_splat_i(m * 8 + i)], tot,
                               mask=iota == 0)

    pltpu.sync_copy(totmem, shared.at[pl.ds(wid * _SPW, _SPW)])
    plsc.subcore_barrier()

    @pl.when(wid == 0)
    def _final():
        pltpu.sync_copy(shared, finmem)
        mv = finmem[pl.ds(0, 16)]
        for b in range(1, 32):
            mv = jnp.maximum(mv, finmem[pl.ds(b * 16, 16)])
        mspl = _allmax(mv, iota)
        sacc = jnp.zeros((16,), dtype=F32)
        for b in range(32):
            sacc = sacc + jnp.exp(finmem[pl.ds(b * 16, 16)] - mspl)
        outmem[...] = mspl + _vlog(_allsum(sacc, iota))
        pltpu.sync_copy(outmem, out_h)


_hmm_sc = functools.partial(
    pl.kernel,
    out_type=jax.ShapeDtypeStruct((16,), F32),
    mesh=plsc.VectorSubcoreMesh(
        core_axis_name="c", subcore_axis_name="s", num_cores=1),
    compiler_params=pltpu.CompilerParams(needs_layout_passes=False),
    scratch_types=[
        pltpu.VMEM((3072,), F32),
        pltpu.VMEM((192,), F32),
        pltpu.VMEM((32,), F32),
        pltpu.VMEM((16,), F32),
        pltpu.VMEM((128,), I32),
        pltpu.VMEM((32,), F32),
        pltpu.VMEM((32,), F32),
        pltpu.VMEM((16,), F32),
        pltpu.VMEM((16,), F32),
        pltpu.VMEM((16,), F32),
        pltpu.VMEM((32,), F32),
        pltpu.VMEM((32,), F32),
        pltpu.VMEM((1664,), F32),
        pltpu.VMEM((32,), F32),
        pltpu.VMEM((512,), F32),
        pltpu.VMEM((16,), F32),
        pltpu.VMEM_SHARED((512,), F32),
        pltpu.SemaphoreType.DMA,
        pltpu.SemaphoreType.DMA,
        pltpu.SemaphoreType.DMA,
        pltpu.SemaphoreType.DMA,
        pltpu.SemaphoreType.DMA,
    ],
)(_sc_body)


def kernel(choice, transition, emission, prior, ys):
    c_pad = jnp.zeros((16,), F32).at[:_I].set(choice.astype(F32))
    t_flat = transition.astype(F32).reshape(-1)
    e_flat = emission.astype(F32).reshape(-1)
    p_pad = jnp.zeros((32,), F32).at[:_I * _S].set(prior.astype(F32).reshape(-1))
    ys32 = ys.astype(I32)
    out = _hmm_sc(c_pad, t_flat, e_flat, p_pad, ys32)
    return out[0]


# speedup vs baseline: 1.2459x; 1.0044x over previous
import functools

import numpy as np

import jax
import jax.numpy as jnp
from jax import lax
from jax.experimental import pallas as pl
from jax.experimental.pallas import tpu as pltpu
from jax.experimental.pallas import tpu_sc as plsc

F32 = np.float32
I32 = np.int32

_I = 3
_S = 8
_A = 128
_T = 128
_NSUB = 16
_SPW = 32
_NROW = 24

_SCALE = F32(2.0 ** 60)
_LN2_120 = F32(120 * 0.6931471805599453)


def _iota16():
    return lax.iota(I32, 16)


def _perm(v, idx):
    return v.at[idx].get(mode="promise_in_bounds")


def _allsum(v, iota):
    for d in (1, 2, 4, 8):
        v = v + _perm(v, iota ^ d)
    return v


def _allmax(v, iota):
    for d in (1, 2, 4, 8):
        v = jnp.maximum(v, _perm(v, iota ^ d))
    return v


def _splat_f(x):
    return jnp.full((16,), x, dtype=F32)


def _splat_i(x):
    return jnp.full((16,), x, dtype=I32)


_LOGP = (0.9999995231628418, -0.49996355175971985, 0.33265241980552673,
         -0.24453333020210266, 0.17659756541252136, -0.10679824650287628,
         0.04365880414843559, -0.008466342464089394)


def _vlog(x):
    bits = plsc.bitcast(x, I32)
    e = ((bits >> 23) & 0xFF) - 127
    m = plsc.bitcast((bits & 0x007FFFFF) | 0x3F800000, F32)
    ef = e.astype(F32)
    f = m - F32(1.0)
    p = F32(_LOGP[-1])
    for c in _LOGP[-2::-1]:
        p = p * f + F32(c)
    return f * p + (ef * F32(0.693359375) + ef * F32(-2.12194440e-4))


def _row_sumexp_8(tref, row, iota):
    idx = _splat_i(row * 8) + (iota & 7)
    v = plsc.load_gather(tref, [idx])
    s = jnp.where(iota < 8, jnp.exp(v), F32(0.0))
    return _allsum(s, iota)


def _scatter1(ref, pos, vec, iota):
    plsc.store_scatter(ref, [_splat_i(pos)], vec, mask=iota == 0)


def _sc_body(c_h, t_h, e_h, p_h, ys_h, out_h,
             ev, tv, pv, cv, ysm, esums, tsums, psums,
             cmem, lzpm, basemem, plvm, etab, totmem, finmem, outmem, shared,
             dsem0, dsem1, dsem2, dsem3, dsem4):
    iota = _iota16()
    wid = lax.axis_index("s")

    cp0 = pltpu.async_copy(c_h, cv, dsem0)
    cp1 = pltpu.async_copy(t_h, tv, dsem1)
    cp2 = pltpu.async_copy(e_h, ev, dsem2)
    cp3 = pltpu.async_copy(p_h, pv, dsem3)
    cp4 = pltpu.async_copy(ys_h, ysm, dsem4)
    cp0.wait()
    cp1.wait()
    cp2.wait()
    cp3.wait()
    cp4.wait()

    cvec = cv[...]
    s_c = _allsum(jnp.where(iota < _I, jnp.exp(cvec), F32(0.0)), iota)
    c_l = cvec - _vlog(s_c)
    cmem[...] = c_l

    one = _splat_f(F32(1.0))
    esums[pl.ds(0, 16)] = one
    esums[pl.ds(16, 16)] = one
    for r in range(0, _NROW, 4):
        accs = [jnp.exp(ev[pl.ds((r + j) * 128, 16)]) for j in range(4)]
        for k in range(1, 8):
            accs = [a + jnp.exp(ev[pl.ds((r + j) * 128 + 16 * k, 16)])
                    for j, a in enumerate(accs)]
        for j in range(4):
            _scatter1(esums, r + j, _allsum(accs[j], iota), iota)

    ts0 = jnp.zeros((16,), dtype=F32)
    ts1 = jnp.zeros((16,), dtype=F32)
    psv = jnp.zeros((16,), dtype=F32)
    for k in range(8):
        ts0 = ts0 + jnp.exp(plsc.load_gather(tv, [iota * 8 + k]))
        idx1 = jnp.minimum((iota + 16) * 8 + k, 191)
        ts1 = ts1 + jnp.exp(plsc.load_gather(tv, [idx1]))
        psv = psv + jnp.exp(plsc.load_gather(pv, [jnp.minimum(iota * 8 + k, 31)]))
    tsums[pl.ds(0, 16)] = ts0
    tsums[pl.ds(16, 16)] = ts1
    lzpm[...] = _vlog(psv)

    plvm[pl.ds(0, 16)] = pv[pl.ds(0, 16)] - plsc.load_gather(lzpm, [iota >> 3])
    plvm[pl.ds(16, 16)] = pv[pl.ds(16, 16)] - plsc.load_gather(lzpm, [_splat_i(2)])

    diag0 = plsc.load_gather(tv, [iota * 8 + (iota & 7)])
    r2 = iota + 16
    didx2 = jnp.minimum(r2 * 8 + (r2 & 7), 191)
    diag1 = plsc.load_gather(tv, [didx2])
    d0 = diag0 - _vlog(tsums[pl.ds(0, 16)])
    d1 = diag1 - _vlog(tsums[pl.ds(16, 16)])
    sc60 = _splat_f(F32(60 * 0.6931471805599453))
    base0 = plsc.load_gather(cmem, [iota >> 3]) + d0 - _vlog(esums[pl.ds(0, 16)])
    base1 = plsc.load_gather(cmem, [_splat_i(2)]) + d1 - _vlog(esums[pl.ds(16, 16)])
    basemem[pl.ds(0, 16)] = base0 + sc60
    basemem[pl.ds(16, 16)] = base1 + sc60

    a_row = wid >> 1
    b_lo = (wid & 1) * 4
    rows = [a_row] + [8 + b_lo + m for m in range(4)] + [16 + n for n in range(8)]

    bspl_rows = [plsc.load_gather(basemem, [_splat_i(row)]) for row in rows]
    ridx_rows = [_splat_i(row * 128) for row in rows]
    for tc in range(8):
        yv = ysm[pl.ds(tc * 16, 16)]
        for rpos in range(13):
            g = plsc.load_gather(ev, [ridx_rows[rpos] + yv])
            etab[pl.ds((tc * 13 + rpos) * 16, 16)] = bspl_rows[rpos] + g

    for j in range(104):
        etab[pl.ds(j * 16, 16)] = jnp.exp(etab[pl.ds(j * 16, 16)])

    pr_rows = [plsc.load_gather(plvm, [_splat_i(row)]) for row in rows]

    lc = _splat_f(_LN2_120)

    for m in range(4):
        accs = [jnp.zeros((16,), dtype=F32) for _ in range(8)]
        for tp in range(4):
            o1 = (2 * tp) * 13 * 16
            o2 = (2 * tp + 1) * 13 * 16
            s1 = etab[pl.ds(o1, 16)] + etab[pl.ds(o1 + (1 + m) * 16, 16)]
            s2 = etab[pl.ds(o2, 16)] + etab[pl.ds(o2 + (1 + m) * 16, 16)]
            ps = [(s1 + etab[pl.ds(o1 + (5 + i) * 16, 16)])
                  * (s2 + etab[pl.ds(o2 + (5 + i) * 16, 16)])
                  for i in range(8)]
            accs = [a + (_vlog(p) - lc) for a, p in zip(accs, ps)]
        for i in range(8):
            tot = (_allsum(accs[i], iota)
                   + pr_rows[0] + pr_rows[1 + m] + pr_rows[5 + i])
            plsc.store_scatter(totmem, [_splat_i(m * 8 + i)], tot,
                               mask=iota == 0)

    v0 = totmem[pl.ds(0, 16)]
    v1 = totmem[pl.ds(16, 16)]
    ml = _allmax(jnp.maximum(v0, v1), iota)
    sl = _allsum(jnp.exp(v0 - ml) + jnp.exp(v1 - ml), iota)
    outmem[...] = jnp.where(iota < 1, ml, sl)
    pltpu.sync_copy(outmem, shared.at[pl.ds(wid * 16, 16)])
    plsc.subcore_barrier()

    @pl.when(wid == 0)
    def _final():
        pltpu.sync_copy(shared.at[pl.ds(0, 256)], finmem.at[pl.ds(0, 256)])
        mlv = plsc.load_gather(finmem, [iota * 16])
        slv = plsc.load_gather(finmem, [iota * 16 + 1])
        mspl = _allmax(mlv, iota)
        s = _allsum(slv * jnp.exp(mlv - mspl), iota)
        outmem[...] = mspl + _vlog(s)
        pltpu.sync_copy(outmem, out_h)


_hmm_sc = functools.partial(
    pl.kernel,
    out_type=jax.ShapeDtypeStruct((16,), F32),
    mesh=plsc.VectorSubcoreMesh(
        core_axis_name="c", subcore_axis_name="s", num_cores=1),
    compiler_params=pltpu.CompilerParams(needs_layout_passes=False),
    scratch_types=[
        pltpu.VMEM((3072,), F32),
        pltpu.VMEM((192,), F32),
        pltpu.VMEM((32,), F32),
        pltpu.VMEM((16,), F32),
        pltpu.VMEM((128,), I32),
        pltpu.VMEM((32,), F32),
        pltpu.VMEM((32,), F32),
        pltpu.VMEM((16,), F32),
        pltpu.VMEM((16,), F32),
        pltpu.VMEM((16,), F32),
        pltpu.VMEM((32,), F32),
        pltpu.VMEM((32,), F32),
        pltpu.VMEM((1664,), F32),
        pltpu.VMEM((32,), F32),
        pltpu.VMEM((512,), F32),
        pltpu.VMEM((16,), F32),
        pltpu.VMEM_SHARED((512,), F32),
        pltpu.SemaphoreType.DMA,
        pltpu.SemaphoreType.DMA,
        pltpu.SemaphoreType.DMA,
        pltpu.SemaphoreType.DMA,
        pltpu.SemaphoreType.DMA,
    ],
)(_sc_body)


def kernel(choice, transition, emission, prior, ys):
    c_pad = jnp.zeros((16,), F32).at[:_I].set(choice.astype(F32))
    t_flat = transition.astype(F32).reshape(-1)
    e_flat = emission.astype(F32).reshape(-1)
    p_pad = jnp.zeros((32,), F32).at[:_I * _S].set(prior.astype(F32).reshape(-1))
    ys32 = ys.astype(I32)
    out = _hmm_sc(c_pad, t_flat, e_flat, p_pad, ys32)
    return out[0]


# speedup vs baseline: 1.2518x; 1.0048x over previous
import functools

import numpy as np

import jax
import jax.numpy as jnp
from jax import lax
from jax.experimental import pallas as pl
from jax.experimental.pallas import tpu as pltpu
from jax.experimental.pallas import tpu_sc as plsc

F32 = np.float32
I32 = np.int32

_I = 3
_S = 8
_A = 128
_T = 128
_NSUB = 16
_SPW = 32
_NROW = 24

_LN2_120 = F32(120 * 0.6931471805599453)


def _iota16():
    return lax.iota(I32, 16)


def _perm(v, idx):
    return v.at[idx].get(mode="promise_in_bounds")


def _allsum(v, iota):
    for d in (1, 2, 4, 8):
        v = v + _perm(v, iota ^ d)
    return v


def _allmax(v, iota):
    for d in (1, 2, 4, 8):
        v = jnp.maximum(v, _perm(v, iota ^ d))
    return v


def _splat_f(x):
    return jnp.full((16,), x, dtype=F32)


def _splat_i(x):
    return jnp.full((16,), x, dtype=I32)


_LOGP = (0.9999995231628418, -0.49996355175971985, 0.33265241980552673,
         -0.24453333020210266, 0.17659756541252136, -0.10679824650287628,
         0.04365880414843559, -0.008466342464089394)


def _vlog(x):
    bits = plsc.bitcast(x, I32)
    e = ((bits >> 23) & 0xFF) - 127
    m = plsc.bitcast((bits & 0x007FFFFF) | 0x3F800000, F32)
    ef = e.astype(F32)
    f = m - F32(1.0)
    p = F32(_LOGP[-1])
    for c in _LOGP[-2::-1]:
        p = p * f + F32(c)
    return f * p + (ef * F32(0.693359375) + ef * F32(-2.12194440e-4))


def _scatter1(ref, pos, vec, iota):
    plsc.store_scatter(ref, [_splat_i(pos)], vec, mask=iota == 0)


def _sc_body(c_h, t_h, e_h, p_h, ys_h, out_h,
             ev, tv, pv, cv, ysm, esums, tsums, psums,
             cmem, lzpm, basemem, plvm, etab, totmem, finmem, outmem, shared,
             dsem0, dsem1, dsem2, dsem3, dsem4):
    iota = _iota16()
    wid = lax.axis_index("s")

    cp0 = pltpu.async_copy(c_h, cv, dsem0)
    cp1 = pltpu.async_copy(t_h, tv, dsem1)
    cp2 = pltpu.async_copy(e_h, ev, dsem2)
    cp3 = pltpu.async_copy(p_h, pv, dsem3)
    cp4 = pltpu.async_copy(ys_h, ysm, dsem4)
    cp0.wait()
    cp1.wait()
    cp2.wait()
    cp3.wait()
    cp4.wait()

    cvec = cv[...]
    s_c = _allsum(jnp.where(iota < _I, jnp.exp(cvec), F32(0.0)), iota)
    c_l = cvec - _vlog(s_c)
    cmem[...] = c_l

    one = _splat_f(F32(1.0))
    esums[pl.ds(0, 16)] = one
    esums[pl.ds(16, 16)] = one
    for r in range(0, _NROW, 4):
        accs = [jnp.exp(ev[pl.ds((r + j) * 128, 16)]) for j in range(4)]
        for k in range(1, 8):
            accs = [a + jnp.exp(ev[pl.ds((r + j) * 128 + 16 * k, 16)])
                    for j, a in enumerate(accs)]
        for j in range(4):
            _scatter1(esums, r + j, _allsum(accs[j], iota), iota)

    ts0 = jnp.zeros((16,), dtype=F32)
    ts1 = jnp.zeros((16,), dtype=F32)
    psv = jnp.zeros((16,), dtype=F32)
    for k in range(8):
        ts0 = ts0 + jnp.exp(plsc.load_gather(tv, [iota * 8 + k]))
        idx1 = jnp.minimum((iota + 16) * 8 + k, 191)
        ts1 = ts1 + jnp.exp(plsc.load_gather(tv, [idx1]))
        psv = psv + jnp.exp(plsc.load_gather(pv, [jnp.minimum(iota * 8 + k, 31)]))
    tsums[pl.ds(0, 16)] = ts0
    tsums[pl.ds(16, 16)] = ts1
    lzpm[...] = _vlog(psv)

    plvm[pl.ds(0, 16)] = pv[pl.ds(0, 16)] - plsc.load_gather(lzpm, [iota >> 3])
    plvm[pl.ds(16, 16)] = pv[pl.ds(16, 16)] - plsc.load_gather(lzpm, [_splat_i(2)])

    diag0 = plsc.load_gather(tv, [iota * 8 + (iota & 7)])
    r2 = iota + 16
    didx2 = jnp.minimum(r2 * 8 + (r2 & 7), 191)
    diag1 = plsc.load_gather(tv, [didx2])
    d0 = diag0 - _vlog(tsums[pl.ds(0, 16)])
    d1 = diag1 - _vlog(tsums[pl.ds(16, 16)])
    sc60 = _splat_f(F32(60 * 0.6931471805599453))
    base0 = plsc.load_gather(cmem, [iota >> 3]) + d0 - _vlog(esums[pl.ds(0, 16)])
    base1 = plsc.load_gather(cmem, [_splat_i(2)]) + d1 - _vlog(esums[pl.ds(16, 16)])
    basemem[pl.ds(0, 16)] = base0 + sc60
    basemem[pl.ds(16, 16)] = base1 + sc60

    a_row = wid >> 1
    b_lo = (wid & 1) * 4
    rows = [a_row] + [8 + b_lo + m for m in range(4)] + [16 + n for n in range(8)]

    bspl_rows = [plsc.load_gather(basemem, [_splat_i(row)]) for row in rows]
    ridx_rows = [_splat_i(row * 128) for row in rows]
    for tc in range(8):
        yv = ysm[pl.ds(tc * 16, 16)]
        for rpos in range(13):
            g = plsc.load_gather(ev, [ridx_rows[rpos] + yv])
            etab[pl.ds((tc * 13 + rpos) * 16, 16)] = bspl_rows[rpos] + g

    for j in range(104):
        etab[pl.ds(j * 16, 16)] = jnp.exp(etab[pl.ds(j * 16, 16)])

    pr_rows = [plsc.load_gather(plvm, [_splat_i(row)]) for row in rows]

    lc = _splat_f(_LN2_120)

    for m in range(4):
        accs = [jnp.zeros((16,), dtype=F32) for _ in range(8)]
        for tp in range(4):
            o1 = (2 * tp) * 13 * 16
            o2 = (2 * tp + 1) * 13 * 16
            s1 = etab[pl.ds(o1, 16)] + etab[pl.ds(o1 + (1 + m) * 16, 16)]
            s2 = etab[pl.ds(o2, 16)] + etab[pl.ds(o2 + (1 + m) * 16, 16)]
            ps = [(s1 + etab[pl.ds(o1 + (5 + i) * 16, 16)])
                  * (s2 + etab[pl.ds(o2 + (5 + i) * 16, 16)])
                  for i in range(8)]
            accs = [a + (_vlog(p) - lc) for a, p in zip(accs, ps)]
        for i in range(8):
            tot = (_allsum(accs[i], iota)
                   + pr_rows[0] + pr_rows[1 + m] + pr_rows[5 + i])
            plsc.store_scatter(totmem, [_splat_i(m * 8 + i)], tot,
                               mask=iota == 0)

    v0 = totmem[pl.ds(0, 16)]
    v1 = totmem[pl.ds(16, 16)]
    ml = _allmax(jnp.maximum(v0, v1), iota)
    sl = _allsum(jnp.exp(v0 - ml) + jnp.exp(v1 - ml), iota)
    outmem[...] = jnp.where(iota < 1, ml, sl)
    pltpu.sync_copy(outmem, shared.at[pl.ds(wid * 16, 16)])
    plsc.subcore_barrier()

    @pl.when(wid == 0)
    def _final():
        pltpu.sync_copy(shared.at[pl.ds(0, 256)], finmem.at[pl.ds(0, 256)])
        mlv = plsc.load_gather(finmem, [iota * 16])
        slv = plsc.load_gather(finmem, [iota * 16 + 1])
        mspl = _allmax(mlv, iota)
        s = _allsum(slv * jnp.exp(mlv - mspl), iota)
        outmem[...] = mspl + _vlog(s)
        pltpu.sync_copy(outmem, out_h)


_hmm_sc = functools.partial(
    pl.kernel,
    out_type=jax.ShapeDtypeStruct((16,), F32),
    mesh=plsc.VectorSubcoreMesh(
        core_axis_name="c", subcore_axis_name="s", num_cores=1),
    compiler_params=pltpu.CompilerParams(needs_layout_passes=False),
    scratch_types=[
        pltpu.VMEM((3072,), F32),
        pltpu.VMEM((192,), F32),
        pltpu.VMEM((32,), F32),
        pltpu.VMEM((16,), F32),
        pltpu.VMEM((128,), I32),
        pltpu.VMEM((32,), F32),
        pltpu.VMEM((32,), F32),
        pltpu.VMEM((16,), F32),
        pltpu.VMEM((16,), F32),
        pltpu.VMEM((16,), F32),
        pltpu.VMEM((32,), F32),
        pltpu.VMEM((32,), F32),
        pltpu.VMEM((1664,), F32),
        pltpu.VMEM((32,), F32),
        pltpu.VMEM((512,), F32),
        pltpu.VMEM((16,), F32),
        pltpu.VMEM_SHARED((512,), F32),
        pltpu.SemaphoreType.DMA,
        pltpu.SemaphoreType.DMA,
        pltpu.SemaphoreType.DMA,
        pltpu.SemaphoreType.DMA,
        pltpu.SemaphoreType.DMA,
    ],
)(_sc_body)


def kernel(choice, transition, emission, prior, ys):
    c_pad = jnp.zeros((16,), F32).at[:_I].set(choice.astype(F32))
    t_flat = transition.astype(F32).reshape(-1)
    e_flat = emission.astype(F32).reshape(-1)
    p_pad = jnp.zeros((32,), F32).at[:_I * _S].set(prior.astype(F32).reshape(-1))
    ys32 = ys.astype(I32)
    out = _hmm_sc(c_pad, t_flat, e_flat, p_pad, ys32)
    return out[0]
